# Initial kernel scaffold; baseline (speedup 1.0000x reference)
import jax
import jax.numpy as jnp
from jax.experimental import pallas as pl


def kernel(token_ids, indexing):
    raise NotImplementedError("write your pallas kernel here")


# speedup vs baseline: 1.6836x; 1.6836x over previous
import jax
import jax.numpy as jnp
from jax import lax
from jax.experimental import pallas as pl
from jax.experimental.pallas import tpu as pltpu
from jax.experimental.pallas import tpu_sc as plsc

NUM_ROWS = 16384 * 50
DIM = 64
NUM_WORKERS = 32
CHUNK = 128
CHUNKS_PER_W = NUM_ROWS // (NUM_WORKERS * CHUNK)


def _emb_body(table_hbm, idx_hbm, out_hbm, idx_v, rows_v, gsem):
    wid = lax.axis_index("s") * 2 + lax.axis_index("c")
    base = wid * CHUNKS_PER_W
    pltpu.sync_copy(idx_hbm.at[pl.ds(base, CHUNKS_PER_W)], idx_v)

    def step(i, carry):
        pltpu.async_copy(table_hbm.at[idx_v.at[i]], rows_v, gsem).wait()
        pltpu.sync_copy(rows_v, out_hbm.at[base + i])
        return carry

    lax.fori_loop(0, CHUNKS_PER_W, step, 0)


@jax.jit
def kernel(token_ids, indexing):
    idx_flat = token_ids.reshape(NUM_ROWS // CHUNK, CHUNK)
    mesh = plsc.VectorSubcoreMesh(core_axis_name="c", subcore_axis_name="s")
    out = pl.kernel(
        _emb_body,
        out_type=jax.ShapeDtypeStruct((NUM_ROWS // CHUNK, CHUNK, DIM), jnp.float32),
        mesh=mesh,
        scratch_types=[
            pltpu.VMEM((CHUNKS_PER_W, CHUNK), jnp.int32),
            pltpu.VMEM((CHUNK, DIM), jnp.float32),
            pltpu.SemaphoreType.DMA,
        ],
        compiler_params=pltpu.CompilerParams(use_tc_tiling_on_sc=False),
    )(indexing, idx_flat)
    return out.reshape(token_ids.shape + (DIM,))


# speedup vs baseline: 1.8893x; 1.1222x over previous
import jax
import jax.numpy as jnp
from jax import lax
from jax.experimental import pallas as pl
from jax.experimental.pallas import tpu as pltpu
from jax.experimental.pallas import tpu_sc as plsc

NUM_ROWS = 16384 * 50
DIM = 64
NUM_WORKERS = 32
CHUNK = 128
CHUNKS_PER_W = NUM_ROWS // (NUM_WORKERS * CHUNK)
NBUF = 4
GROUPS = CHUNKS_PER_W // NBUF - 1


def _emb_body(table_hbm, idx_hbm, out_hbm, idx_v, rows_v, *sems):
    gsem = sems[:NBUF]
    wsem = sems[NBUF:]
    wid = lax.axis_index("s") * 2 + lax.axis_index("c")
    base = wid * CHUNKS_PER_W
    pltpu.sync_copy(idx_hbm.at[pl.ds(base, CHUNKS_PER_W)], idx_v)

    def gather(chunk, b):
        pltpu.async_copy(table_hbm.at[idx_v.at[chunk]], rows_v.at[b], gsem[b])

    for b in range(NBUF):
        gather(b, b)

    def group(g, carry):
        for b in range(NBUF):
            chunk = g * NBUF + b
            pltpu.make_async_copy(table_hbm.at[idx_v.at[chunk]],
                                  rows_v.at[b], gsem[b]).wait()
            pltpu.async_copy(rows_v.at[b], out_hbm.at[base + chunk], wsem[b])
            pltpu.make_async_copy(rows_v.at[b], out_hbm.at[base + chunk],
                                  wsem[b]).wait()
            gather(chunk + NBUF, b)
        return carry

    lax.fori_loop(0, GROUPS, group, 0)

    for b in range(NBUF):
        chunk = GROUPS * NBUF + b
        pltpu.make_async_copy(table_hbm.at[idx_v.at[chunk]],
                              rows_v.at[b], gsem[b]).wait()
        pltpu.async_copy(rows_v.at[b], out_hbm.at[base + chunk], wsem[b])
    for b in range(NBUF):
        chunk = GROUPS * NBUF + b
        pltpu.make_async_copy(rows_v.at[b], out_hbm.at[base + chunk],
                              wsem[b]).wait()


@jax.jit
def kernel(token_ids, indexing):
    idx_flat = token_ids.reshape(NUM_ROWS // CHUNK, CHUNK)
    mesh = plsc.VectorSubcoreMesh(core_axis_name="c", subcore_axis_name="s")
    out = pl.kernel(
        _emb_body,
        out_type=jax.ShapeDtypeStruct((NUM_ROWS // CHUNK, CHUNK, DIM), jnp.float32),
        mesh=mesh,
        scratch_types=[
            pltpu.VMEM((CHUNKS_PER_W, CHUNK), jnp.int32),
            pltpu.VMEM((NBUF, CHUNK, DIM), jnp.float32),
        ] + [pltpu.SemaphoreType.DMA] * (2 * NBUF),
        compiler_params=pltpu.CompilerParams(use_tc_tiling_on_sc=False),
    )(indexing, idx_flat)
    return out.reshape(token_ids.shape + (DIM,))
